# baseline (speedup 1.0000x reference)
import functools

import jax
import jax.numpy as jnp
from jax import lax
from jax.experimental import pallas as pl
from jax.experimental.pallas import tpu as pltpu
from jax.experimental.pallas import tpu_sc as plsc

_NUM_CLASSES = 100000
_FEAT = 128
_BATCH = 16384

_NC = 2
_NS = 16
_NW = _NC * _NS
_RPW = _BATCH // _NW
_CHUNK = 64
_NCHUNK = _RPW // _CHUNK
_HIST_PER_SUB = 6272
_HIST_PAD = _HIST_PER_SUB * _NS
_LBL_PER_SUB = _BATCH // _NS
_ZCHUNK = _HIST_PER_SUB // 4


def _sc_body(y_hbm, feat_hbm, centers_hbm, out_hbm,
             hist_sh, zeros_v, ones_v, lbl_v, own_v, cnt_v, w_v,
             wbc_v, f_v, g_v, part_v, semg0, semg1, semf0, semf1):
    cid = lax.axis_index("c")
    sid = lax.axis_index("s")
    wid = sid * _NC + cid
    semg = [semg0, semg1]
    semf = [semf0, semf1]

    pltpu.sync_copy(y_hbm.at[pl.ds(wid * _RPW, _RPW)], own_v)

    pend_g = [None, None]
    pend_f = [None, None]
    pend_g[0] = pltpu.async_copy(
        centers_hbm.at[own_v.at[pl.ds(0, _CHUNK)]], g_v.at[0], semg[0])
    pend_f[0] = pltpu.async_copy(
        feat_hbm.at[pl.ds(wid * _RPW, _CHUNK)], f_v.at[0], semf[0])

    def _fill_zeros(i, _):
        zeros_v[pl.ds(i * 16, 16)] = jnp.zeros((16,), jnp.float32)
        return 0
    lax.fori_loop(0, _ZCHUNK // 16, _fill_zeros, 0, unroll=8)
    for j in range(128 // 16):
        ones_v[pl.ds(j * 16, 16)] = jnp.ones((16,), jnp.float32)

    zcopies = [
        pltpu.async_copy(
            zeros_v,
            hist_sh.at[pl.ds(sid * _HIST_PER_SUB + j * _ZCHUNK, _ZCHUNK)],
            semf[1])
        for j in range(_HIST_PER_SUB // _ZCHUNK)
    ]
    lcopies = [
        pltpu.async_copy(
            y_hbm.at[pl.ds(sid * _LBL_PER_SUB + j * 128, 128)],
            lbl_v.at[j], semg[1])
        for j in range(_LBL_PER_SUB // 128)
    ]
    for d in zcopies + lcopies:
        d.wait()

    plsc.subcore_barrier()

    scopies = [
        pltpu.async_copy(ones_v, hist_sh.at[lbl_v.at[j]], semg[1], add=True)
        for j in range(_LBL_PER_SUB // 128)
    ]
    for d in scopies:
        d.wait()

    plsc.subcore_barrier()

    ccopies = [
        pltpu.async_copy(hist_sh.at[own_v.at[pl.ds(c * 128, 128)]],
                         cnt_v.at[pl.ds(c * 128, 128)], semg[1])
        for c in range(_RPW // 128)
    ]
    for d in ccopies:
        d.wait()

    for jj in range(_RPW // 16):
        sl = pl.ds(jj * 16, 16)
        w_v[sl] = 1.0 / (cnt_v[sl] + 1.0)

    def _wrow(rg, _):
        w16 = w_v[pl.ds(rg * 16, 16)]
        for k in range(16):
            wbc_v[rg * 16 + k, :] = jnp.broadcast_to(w16[k], (16,))
        return 0
    lax.fori_loop(0, _RPW // 16, _wrow, 0)

    pltpu.async_copy(
        centers_hbm.at[own_v.at[pl.ds(_CHUNK, _CHUNK)]], g_v.at[1], semg[1])
    pltpu.async_copy(
        feat_hbm.at[pl.ds(wid * _RPW + _CHUNK, _CHUNK)], f_v.at[1], semf[1])

    def _chunk_pair(p, acc):
        for b in range(2):
            c = 2 * p + b
            pltpu.make_async_copy(
                centers_hbm.at[own_v.at[pl.ds(0, _CHUNK)]],
                g_v.at[b], semg[b]).wait()
            pltpu.make_async_copy(
                feat_hbm.at[pl.ds(0, _CHUNK)], f_v.at[b], semf[b]).wait()

            def _row(r, acc2):
                wb = wbc_v[c * _CHUNK + r, :]
                new = []
                for jj in range(_FEAT // 16):
                    sl = pl.ds(jj * 16, 16)
                    a, bb, cc, ff, ss = acc2[5 * jj:5 * jj + 5]
                    f = f_v[b, r, sl]
                    g = g_v[b, r, sl]
                    t1 = f * f
                    t2 = g * g
                    t3 = f * g
                    new.extend((a + wb * t1, bb + wb * t3, cc + wb * t2,
                                ff + t1, ss + t2))
                return tuple(new)

            acc = lax.fori_loop(0, _CHUNK, _row, acc)

            @pl.when(c + 2 < _NCHUNK)
            def _():
                nc = pl.multiple_of((c + 2) * _CHUNK, _CHUNK)
                pltpu.async_copy(
                    centers_hbm.at[own_v.at[pl.ds(nc, _CHUNK)]],
                    g_v.at[b], semg[b])
                pltpu.async_copy(
                    feat_hbm.at[pl.ds(wid * _RPW + nc, _CHUNK)],
                    f_v.at[b], semf[b])
        return acc

    z = jnp.zeros((16,), jnp.float32)
    acc = lax.fori_loop(0, _NCHUNK // 2, _chunk_pair,
                        (z,) * (5 * _FEAT // 16))
    for jj in range(_FEAT // 16):
        a, b, cc, ff, ss = acc[5 * jj:5 * jj + 5]
        o = jj * 16
        part_v[pl.ds(o, 16)] = a
        part_v[pl.ds(_FEAT + o, 16)] = b
        part_v[pl.ds(2 * _FEAT + o, 16)] = cc
        part_v[pl.ds(3 * _FEAT + o, 16)] = ff
        part_v[pl.ds(4 * _FEAT + o, 16)] = ss

    pltpu.sync_copy(part_v, out_hbm.at[wid])


@functools.partial(jax.jit, static_argnames=())
def _sc_partials(y, feat, centers):
    mesh = plsc.VectorSubcoreMesh(core_axis_name="c", subcore_axis_name="s")
    return pl.kernel(
        _sc_body,
        out_type=jax.ShapeDtypeStruct((_NW, 5 * _FEAT), jnp.float32),
        mesh=mesh,
        scratch_types=[
            pltpu.VMEM_SHARED((_HIST_PAD,), jnp.float32),
            pltpu.VMEM((_ZCHUNK,), jnp.float32),
            pltpu.VMEM((128,), jnp.float32),
            pltpu.VMEM((_LBL_PER_SUB // 128, 128), jnp.int32),
            pltpu.VMEM((_RPW,), jnp.int32),
            pltpu.VMEM((_RPW,), jnp.float32),
            pltpu.VMEM((_RPW,), jnp.float32),
            pltpu.VMEM((_RPW, 16), jnp.float32),
            pltpu.VMEM((2, _CHUNK, _FEAT), jnp.float32),
            pltpu.VMEM((2, _CHUNK, _FEAT), jnp.float32),
            pltpu.VMEM((5 * _FEAT,), jnp.float32),
            pltpu.SemaphoreType.DMA,
            pltpu.SemaphoreType.DMA,
            pltpu.SemaphoreType.DMA,
            pltpu.SemaphoreType.DMA,
        ],
        name="center_loss_sc",
    )(y, feat, centers)


def _tc_finish_body(p_ref, o_ref):
    p = p_ref[...]
    s = jnp.sum(p, axis=0, keepdims=True)
    a = s[:, 0:_FEAT]
    b = s[:, _FEAT:2 * _FEAT]
    c = s[:, 2 * _FEAT:3 * _FEAT]
    f = s[:, 3 * _FEAT:4 * _FEAT]
    g = s[:, 4 * _FEAT:5 * _FEAT]
    nf = jnp.sqrt(f)
    ng = jnp.sqrt(g)
    maxnorm = 1e-05
    sf = jnp.where(nf > maxnorm, maxnorm / jnp.maximum(nf, 1e-30), 1.0) * 1e5
    sc = jnp.where(ng > maxnorm, maxnorm / jnp.maximum(ng, 1e-30), 1.0) * 1e5
    loss = 0.5 * jnp.sum(sf * sf * a - 2.0 * (sf * sc) * b + sc * sc * c)
    o_ref[0, 0] = loss


def kernel(y, feat, centers):
    part = _sc_partials(y.astype(jnp.int32), feat, centers)
    out = pl.pallas_call(
        _tc_finish_body,
        out_shape=jax.ShapeDtypeStruct((1, 1), jnp.float32),
        out_specs=pl.BlockSpec(memory_space=pltpu.SMEM),
    )(part)
    return out[0, 0]


# speedup vs baseline: 1.1428x; 1.1428x over previous
import functools

import jax
import jax.numpy as jnp
from jax import lax
from jax.experimental import pallas as pl
from jax.experimental.pallas import tpu as pltpu
from jax.experimental.pallas import tpu_sc as plsc

_NUM_CLASSES = 100000
_FEAT = 128
_BATCH = 16384

_NC = 2
_NS = 16
_NW = _NC * _NS
_RPW = _BATCH // _NW
_CHUNK = 64
_NCHUNK = _RPW // _CHUNK
_HIST_PER_SUB = 6272
_HIST_PAD = _HIST_PER_SUB * _NS
_LBL_PER_SUB = _BATCH // _NS
_ZCHUNK = _HIST_PER_SUB // 4


def _sc_body(y_hbm, feat_hbm, centers_hbm, out_hbm,
             hist_sh, zeros_v, ones_v, lbl_v, own_v, cnt_v, w_v,
             wbc_v, f_v, g_v, part_v, semg0, semg1, semf0, semf1):
    cid = lax.axis_index("c")
    sid = lax.axis_index("s")
    wid = sid * _NC + cid
    semg = [semg0, semg1]
    semf = [semf0, semf1]

    pltpu.sync_copy(y_hbm.at[pl.ds(wid * _RPW, _RPW)], own_v)

    pend_g = [None, None]
    pend_f = [None, None]
    pend_g[0] = pltpu.async_copy(
        centers_hbm.at[own_v.at[pl.ds(0, _CHUNK)]], g_v.at[0], semg[0])
    pend_f[0] = pltpu.async_copy(
        feat_hbm.at[pl.ds(wid * _RPW, _CHUNK)], f_v.at[0], semf[0])

    def _fill_zeros(i, _):
        zeros_v[pl.ds(i * 16, 16)] = jnp.zeros((16,), jnp.float32)
        return 0
    lax.fori_loop(0, _ZCHUNK // 16, _fill_zeros, 0, unroll=8)
    for j in range(128 // 16):
        ones_v[pl.ds(j * 16, 16)] = jnp.ones((16,), jnp.float32)

    zcopies = [
        pltpu.async_copy(
            zeros_v,
            hist_sh.at[pl.ds(sid * _HIST_PER_SUB + j * _ZCHUNK, _ZCHUNK)],
            semf[1])
        for j in range(_HIST_PER_SUB // _ZCHUNK)
    ]
    lcopies = [
        pltpu.async_copy(
            y_hbm.at[pl.ds(sid * _LBL_PER_SUB + j * 128, 128)],
            lbl_v.at[j], semg[1])
        for j in range(_LBL_PER_SUB // 128)
    ]
    for d in zcopies + lcopies:
        d.wait()

    plsc.subcore_barrier()

    scopies = [
        pltpu.async_copy(ones_v, hist_sh.at[lbl_v.at[j]], semg[1], add=True)
        for j in range(_LBL_PER_SUB // 128)
    ]
    for d in scopies:
        d.wait()

    plsc.subcore_barrier()

    ccopies = [
        pltpu.async_copy(hist_sh.at[own_v.at[pl.ds(c * 128, 128)]],
                         cnt_v.at[pl.ds(c * 128, 128)], semg[1])
        for c in range(_RPW // 128)
    ]
    for d in ccopies:
        d.wait()

    for jj in range(_RPW // 16):
        sl = pl.ds(jj * 16, 16)
        w_v[sl] = 1.0 / (cnt_v[sl] + 1.0)

    def _wrow(rg, _):
        w16 = w_v[pl.ds(rg * 16, 16)]
        for k in range(16):
            wbc_v[rg * 16 + k, :] = jnp.broadcast_to(w16[k], (16,))
        return 0
    lax.fori_loop(0, _RPW // 16, _wrow, 0)

    acc = (jnp.zeros((16,), jnp.float32),) * (5 * _FEAT // 16)
    for c in range(_NCHUNK):
        cur = c % 2
        nxt = (c + 1) % 2
        if c + 1 < _NCHUNK:
            pend_g[nxt] = pltpu.async_copy(
                centers_hbm.at[own_v.at[pl.ds((c + 1) * _CHUNK, _CHUNK)]],
                g_v.at[nxt], semg[nxt])
            pend_f[nxt] = pltpu.async_copy(
                feat_hbm.at[pl.ds(wid * _RPW + (c + 1) * _CHUNK, _CHUNK)],
                f_v.at[nxt], semf[nxt])
        pend_g[cur].wait()
        pend_f[cur].wait()

        def _row(r, acc2):
            wb = wbc_v[c * _CHUNK + r, :]
            new = []
            for jj in range(_FEAT // 16):
                sl = pl.ds(jj * 16, 16)
                a, bb, cc, ff, ss = acc2[5 * jj:5 * jj + 5]
                f = f_v[cur, r, sl]
                g = g_v[cur, r, sl]
                t1 = f * f
                t2 = g * g
                t3 = f * g
                new.extend((a + wb * t1, bb + wb * t3, cc + wb * t2,
                            ff + t1, ss + t2))
            return tuple(new)

        acc = lax.fori_loop(0, _CHUNK, _row, acc)

    for jj in range(_FEAT // 16):
        a, b, cc, ff, ss = acc[5 * jj:5 * jj + 5]
        o = jj * 16
        part_v[pl.ds(o, 16)] = a
        part_v[pl.ds(_FEAT + o, 16)] = b
        part_v[pl.ds(2 * _FEAT + o, 16)] = cc
        part_v[pl.ds(3 * _FEAT + o, 16)] = ff
        part_v[pl.ds(4 * _FEAT + o, 16)] = ss

    pltpu.sync_copy(part_v, out_hbm.at[wid])


@functools.partial(jax.jit, static_argnames=())
def _sc_partials(y, feat, centers):
    mesh = plsc.VectorSubcoreMesh(core_axis_name="c", subcore_axis_name="s")
    return pl.kernel(
        _sc_body,
        out_type=jax.ShapeDtypeStruct((_NW, 5 * _FEAT), jnp.float32),
        mesh=mesh,
        scratch_types=[
            pltpu.VMEM_SHARED((_HIST_PAD,), jnp.float32),
            pltpu.VMEM((_ZCHUNK,), jnp.float32),
            pltpu.VMEM((128,), jnp.float32),
            pltpu.VMEM((_LBL_PER_SUB // 128, 128), jnp.int32),
            pltpu.VMEM((_RPW,), jnp.int32),
            pltpu.VMEM((_RPW,), jnp.float32),
            pltpu.VMEM((_RPW,), jnp.float32),
            pltpu.VMEM((_RPW, 16), jnp.float32),
            pltpu.VMEM((2, _CHUNK, _FEAT), jnp.float32),
            pltpu.VMEM((2, _CHUNK, _FEAT), jnp.float32),
            pltpu.VMEM((5 * _FEAT,), jnp.float32),
            pltpu.SemaphoreType.DMA,
            pltpu.SemaphoreType.DMA,
            pltpu.SemaphoreType.DMA,
            pltpu.SemaphoreType.DMA,
        ],
        name="center_loss_sc",
    )(y, feat, centers)


def _tc_finish_body(p_ref, o_ref):
    p = p_ref[...]
    s = jnp.sum(p, axis=0, keepdims=True)
    a = s[:, 0:_FEAT]
    b = s[:, _FEAT:2 * _FEAT]
    c = s[:, 2 * _FEAT:3 * _FEAT]
    f = s[:, 3 * _FEAT:4 * _FEAT]
    g = s[:, 4 * _FEAT:5 * _FEAT]
    nf = jnp.sqrt(f)
    ng = jnp.sqrt(g)
    maxnorm = 1e-05
    sf = jnp.where(nf > maxnorm, maxnorm / jnp.maximum(nf, 1e-30), 1.0) * 1e5
    sc = jnp.where(ng > maxnorm, maxnorm / jnp.maximum(ng, 1e-30), 1.0) * 1e5
    loss = 0.5 * jnp.sum(sf * sf * a - 2.0 * (sf * sc) * b + sc * sc * c)
    o_ref[0, 0] = loss


def kernel(y, feat, centers):
    part = _sc_partials(y.astype(jnp.int32), feat, centers)
    out = pl.pallas_call(
        _tc_finish_body,
        out_shape=jax.ShapeDtypeStruct((1, 1), jnp.float32),
        out_specs=pl.BlockSpec(memory_space=pltpu.SMEM),
    )(part)
    return out[0, 0]
